# baseline (speedup 1.0000x reference)
import jax
import jax.numpy as jnp
from jax.experimental import pallas as pl
from jax.experimental.pallas import tpu as pltpu

T = 128
D = 1024
F = 4096
E = 8
FB = 2048


def _moe_kernel(x_ref, wg_ref, w1_ref, w3_ref, w2_ref, out_ref, gates_ref):
    e = pl.program_id(0)
    f = pl.program_id(1)

    @pl.when(jnp.logical_and(e == 0, f == 0))
    def _init():
        x = x_ref[...]
        logits = jnp.dot(x, wg_ref[...], preferred_element_type=jnp.float32)
        m = jnp.max(logits, axis=1, keepdims=True)
        p = jnp.exp(logits - m)
        p = p / jnp.sum(p, axis=1, keepdims=True)
        iota = jax.lax.broadcasted_iota(jnp.int32, (T, E), 1)
        v1 = jnp.max(p, axis=1, keepdims=True)
        i1 = jnp.min(jnp.where(p == v1, iota, E), axis=1, keepdims=True)
        mask1 = iota == i1
        p2 = jnp.where(mask1, -1.0, p)
        v2 = jnp.max(p2, axis=1, keepdims=True)
        i2 = jnp.min(jnp.where(p2 == v2, iota, E), axis=1, keepdims=True)
        mask2 = iota == i2
        s = v1 + v2
        gates_ref[...] = (jnp.where(mask1, v1, 0.0) +
                          jnp.where(mask2, v2, 0.0)) / s
        out_ref[...] = jnp.zeros_like(out_ref)

    x = x_ref[...]
    h1 = jnp.dot(x, w1_ref[0], preferred_element_type=jnp.float32)
    h3 = jnp.dot(x, w3_ref[0], preferred_element_type=jnp.float32)
    h = (h1 * jax.lax.logistic(h1)) * h3
    iota = jax.lax.broadcasted_iota(jnp.int32, (T, E), 1)
    g = jnp.sum(jnp.where(iota == e, gates_ref[...], 0.0), axis=1,
                keepdims=True)
    out_ref[...] += jnp.dot(h * g, w2_ref[0],
                            preferred_element_type=jnp.float32)


def kernel(hidden_states, Wg, W1, W3, W2):
    x = hidden_states.reshape(-1, hidden_states.shape[-1])
    nf = F // FB
    return pl.pallas_call(
        _moe_kernel,
        grid=(E, nf),
        in_specs=[
            pl.BlockSpec((T, D), lambda e, f: (0, 0)),
            pl.BlockSpec((D, E), lambda e, f: (0, 0)),
            pl.BlockSpec((1, D, FB), lambda e, f: (e, 0, f)),
            pl.BlockSpec((1, D, FB), lambda e, f: (e, 0, f)),
            pl.BlockSpec((1, FB, D), lambda e, f: (e, f, 0)),
        ],
        out_specs=pl.BlockSpec((T, D), lambda e, f: (0, 0)),
        out_shape=jax.ShapeDtypeStruct((T, D), jnp.float32),
        scratch_shapes=[pltpu.VMEM((T, E), jnp.float32)],
    )(x, Wg, W1, W3, W2)


# speedup vs baseline: 1.0182x; 1.0182x over previous
import jax
import jax.numpy as jnp
from jax.experimental import pallas as pl
from jax.experimental.pallas import tpu as pltpu

T = 128
D = 1024
F = 4096
E = 8
FB = 1024


def _moe_kernel(x_ref, wg_ref, w1_ref, w3_ref, w2_ref, out_ref, gates_ref):
    e = pl.program_id(0)
    f = pl.program_id(1)

    @pl.when(jnp.logical_and(e == 0, f == 0))
    def _init():
        x = x_ref[...]
        logits = jnp.dot(x, wg_ref[...], preferred_element_type=jnp.float32)
        m = jnp.max(logits, axis=1, keepdims=True)
        p = jnp.exp(logits - m)
        p = p / jnp.sum(p, axis=1, keepdims=True)
        iota = jax.lax.broadcasted_iota(jnp.int32, (T, E), 1)
        v1 = jnp.max(p, axis=1, keepdims=True)
        i1 = jnp.min(jnp.where(p == v1, iota, E), axis=1, keepdims=True)
        mask1 = iota == i1
        p2 = jnp.where(mask1, -1.0, p)
        v2 = jnp.max(p2, axis=1, keepdims=True)
        i2 = jnp.min(jnp.where(p2 == v2, iota, E), axis=1, keepdims=True)
        mask2 = iota == i2
        s = v1 + v2
        gates_ref[...] = (jnp.where(mask1, v1, 0.0) +
                          jnp.where(mask2, v2, 0.0)) / s
        out_ref[...] = jnp.zeros_like(out_ref)

    x = x_ref[...].astype(jnp.bfloat16)
    h1 = jnp.dot(x, w1_ref[0].astype(jnp.bfloat16),
                 preferred_element_type=jnp.float32)
    h3 = jnp.dot(x, w3_ref[0].astype(jnp.bfloat16),
                 preferred_element_type=jnp.float32)
    h = (h1 * jax.lax.logistic(h1)) * h3
    iota = jax.lax.broadcasted_iota(jnp.int32, (T, E), 1)
    g = jnp.sum(jnp.where(iota == e, gates_ref[...], 0.0), axis=1,
                keepdims=True)
    out_ref[...] += jnp.dot((h * g).astype(jnp.bfloat16),
                            w2_ref[0].astype(jnp.bfloat16),
                            preferred_element_type=jnp.float32)


def kernel(hidden_states, Wg, W1, W3, W2):
    x = hidden_states.reshape(-1, hidden_states.shape[-1])
    nf = F // FB
    return pl.pallas_call(
        _moe_kernel,
        grid=(E, nf),
        in_specs=[
            pl.BlockSpec((T, D), lambda e, f: (0, 0)),
            pl.BlockSpec((D, E), lambda e, f: (0, 0)),
            pl.BlockSpec((1, D, FB), lambda e, f: (e, 0, f)),
            pl.BlockSpec((1, D, FB), lambda e, f: (e, 0, f)),
            pl.BlockSpec((1, FB, D), lambda e, f: (e, f, 0)),
        ],
        out_specs=pl.BlockSpec((T, D), lambda e, f: (0, 0)),
        out_shape=jax.ShapeDtypeStruct((T, D), jnp.float32),
        scratch_shapes=[pltpu.VMEM((T, E), jnp.float32)],
    )(x, Wg, W1, W3, W2)
